# Initial kernel scaffold; baseline (speedup 1.0000x reference)
import jax
import jax.numpy as jnp
from jax.experimental import pallas as pl


def kernel(seq, proj_w0, proj_b0, embed0, proj_w1, proj_b1, embed1):
    raise NotImplementedError("write your pallas kernel here")


# speedup vs baseline: 1.5134x; 1.5134x over previous
import jax
import jax.numpy as jnp
from jax.experimental import pallas as pl
from jax.experimental.pallas import tpu as pltpu

_TAU = 1.0


def _vq_body(z_ref, w0_ref, b0_ref, e0_ref, u0_ref,
             w1_ref, b1_ref, e1_ref, u1_ref,
             out_ref, loss_ref):
    @pl.when(pl.program_id(0) == 0)
    def _init():
        loss_ref[0, 0] = jnp.float32(0.0)

    z = z_ref[...]
    vocab = w0_ref.shape[1]
    d = e0_ref.shape[1]
    acc = jnp.float32(0.0)
    for idx, (w_ref, b_ref, e_ref, u_ref) in enumerate(
            ((w0_ref, b0_ref, e0_ref, u0_ref),
             (w1_ref, b1_ref, e1_ref, u1_ref))):
        logits = jnp.dot(z, w_ref[...], preferred_element_type=jnp.float32)
        logits = logits + b_ref[...]
        g = -jnp.log(-jnp.log(u_ref[...]))
        y = (logits + g) * (1.0 / _TAU)
        y = y - jnp.max(y, axis=1, keepdims=True)
        ey = jnp.exp(y)
        soft = ey / jnp.sum(ey, axis=1, keepdims=True)
        out_ref[:, idx * d:(idx + 1) * d] = jnp.dot(
            soft, e_ref[...], preferred_element_type=jnp.float32)
        x = logits - jnp.max(logits, axis=1, keepdims=True)
        ex = jnp.exp(x)
        qy = ex / jnp.sum(ex, axis=1, keepdims=True)
        acc = acc + jnp.sum(qy * jnp.log(qy * jnp.float32(vocab) + 1e-10))
    loss_ref[0, 0] += acc


def kernel(seq, proj_w0, proj_b0, embed0, proj_w1, proj_b1, embed1):
    b, l, c = seq.shape
    v = proj_w0.shape[0]
    d = embed0.shape[1]
    tok = b * l

    z = seq.reshape(tok, c)
    base = jax.random.key(42)
    us = []
    for i in range(2):
        u = jax.random.uniform(jax.random.fold_in(base, i), (b, v, l),
                               minval=1e-9, maxval=1.0)
        us.append(jnp.transpose(u, (0, 2, 1)).reshape(tok, v))

    blk = 256
    grid = tok // blk
    out, loss = pl.pallas_call(
        _vq_body,
        grid=(grid,),
        in_specs=[
            pl.BlockSpec((blk, c), lambda i: (i, 0)),
            pl.BlockSpec((c, v), lambda i: (0, 0)),
            pl.BlockSpec((1, v), lambda i: (0, 0)),
            pl.BlockSpec((v, d), lambda i: (0, 0)),
            pl.BlockSpec((blk, v), lambda i: (i, 0)),
            pl.BlockSpec((c, v), lambda i: (0, 0)),
            pl.BlockSpec((1, v), lambda i: (0, 0)),
            pl.BlockSpec((v, d), lambda i: (0, 0)),
            pl.BlockSpec((blk, v), lambda i: (i, 0)),
        ],
        out_specs=[
            pl.BlockSpec((blk, 2 * d), lambda i: (i, 0)),
            pl.BlockSpec((1, 1), lambda i: (0, 0),
                         memory_space=pltpu.SMEM),
        ],
        out_shape=[
            jax.ShapeDtypeStruct((tok, 2 * d), jnp.float32),
            jax.ShapeDtypeStruct((1, 1), jnp.float32),
        ],
    )(z, proj_w0.T, proj_b0.reshape(1, v), embed0, us[0],
      proj_w1.T, proj_b1.reshape(1, v), embed1, us[1])
    return out.reshape(b, l, 2 * d), loss[0, 0] / tok
